# baseline (speedup 1.0000x reference)
import functools

import jax
import jax.numpy as jnp
from jax import lax
from jax.experimental import pallas as pl
from jax.experimental.pallas import tpu as pltpu
from jax.experimental.pallas import tpu_sc as plsc

N_TOKENS = 32768
D = 128
N_MOLS = 16
L = 16
NV = D // L
CH = 128
NBUF = 4

BLOCK = 256
NBLKS = N_TOKENS // BLOCK
TBLK = 2048


def _sc_body(ah, starts_hbm, sizes_hbm, out, buf, acc_v, t0, t1, scope_v,
             shared, *sems):
    c = lax.axis_index("c")
    s = lax.axis_index("s")

    pltpu.sync_copy(starts_hbm, scope_v.at[0, pl.ds(0, N_MOLS)])
    pltpu.sync_copy(sizes_hbm, scope_v.at[1, pl.ds(0, N_MOLS)])

    def _at(row, idx):
        return scope_v[row, pl.ds(idx, L)][0]

    m = 2 * (s // 2) + c
    h = s % 2
    start = _at(0, m)
    size = _at(1, m)
    end = start + size
    p = jnp.minimum(((start + BLOCK - 1) // BLOCK) * BLOCK, end)
    q = jnp.maximum((end // BLOCK) * BLOCK, p)
    a = jnp.where(h == 0, start, q)
    b = jnp.where(h == 0, p, end)
    a8 = (a // 8) * 8
    nch = (b - a8 + CH - 1) // CH

    def _offc(k):
        return jnp.minimum(a8 + k * CH, N_TOKENS - CH)

    def _dma_start(k, buf_ref, sem):
        @pl.when(k < nch)
        def _():
            pltpu.async_copy(ah.at[pl.ds(_offc(k), CH), :], buf_ref, sem)

    def _dma_wait(k, buf_ref, sem):
        @pl.when(k < nch)
        def _():
            pltpu.make_async_copy(ah.at[pl.ds(0, CH), :], buf_ref, sem).wait()

    def _accumulate(k, buf_ref, carry):
        off = a8 + k * CH
        rel_lo = jnp.maximum(a, off) - _offc(k)
        rel_hi = jnp.minimum(b, off + CH) - _offc(k)

        def row_body(r, acc):
            return tuple(acc[kk] + buf_ref[r, pl.ds(kk * L, L)]
                         for kk in range(NV))

        return plsc.parallel_loop(rel_lo, rel_hi, step=1, unroll=8,
                                  carry=carry)(row_body)

    for bslot in range(NBUF):
        _dma_start(bslot, buf.at[bslot], sems[bslot])

    def ring_body(j, carry):
        k0 = NBUF * j
        for bslot in range(NBUF):
            k = k0 + bslot
            _dma_wait(k, buf.at[bslot], sems[bslot])
            carry = _accumulate(k, buf.at[bslot], carry)
            _dma_start(k + NBUF, buf.at[bslot], sems[bslot])
        return carry

    zeros = tuple(jnp.zeros((L,), jnp.float32) for _ in range(NV))
    accs = lax.fori_loop(0, (nch + NBUF - 1) // NBUF, ring_body, zeros)
    for k in range(NV):
        acc_v[pl.ds(k * L, L)] = accs[k]

    pltpu.sync_copy(acc_v, shared.at[s])
    plsc.subcore_barrier()

    @pl.when(s < 8)
    def _():
        pltpu.sync_copy(shared.at[2 * s], t0)
        pltpu.sync_copy(shared.at[2 * s + 1], t1)
        mrow = 2 * s + c
        for k in range(NV):
            sl = pl.ds(k * L, L)
            t0[sl] = t0[sl] + t1[sl]
        pltpu.sync_copy(t0, out.at[pl.ds(mrow * D, D)])


_sc_edges = functools.partial(
    pl.kernel,
    out_type=jax.ShapeDtypeStruct((N_MOLS * D,), jnp.float32),
    mesh=plsc.VectorSubcoreMesh(core_axis_name="c", subcore_axis_name="s"),
    compiler_params=pltpu.CompilerParams(use_tc_tiling_on_sc=False),
    scratch_types=[
        pltpu.VMEM((NBUF, CH, D), jnp.float32),
        pltpu.VMEM((D,), jnp.float32),
        pltpu.VMEM((D,), jnp.float32),
        pltpu.VMEM((D,), jnp.float32),
        pltpu.VMEM((2, 2 * L), jnp.int32),
        pltpu.VMEM_SHARED((N_MOLS, D), jnp.float32),
    ] + [pltpu.SemaphoreType.DMA] * NBUF,
)(_sc_body)


def _tc1_body(x_ref, o_ref):
    x = x_ref[...]
    o_ref[...] = jnp.sum(x.reshape(TBLK // BLOCK, BLOCK, D), axis=1)


def _tc1_block_sums(atom_hiddens):
    return pl.pallas_call(
        _tc1_body,
        out_shape=jax.ShapeDtypeStruct((NBLKS, D), jnp.float32),
        grid=(N_TOKENS // TBLK,),
        in_specs=[pl.BlockSpec((TBLK, D), lambda i: (i, 0))],
        out_specs=pl.BlockSpec((TBLK // BLOCK, D), lambda i: (i, 0)),
    )(atom_hiddens)


def _tc2_body(starts_ref, sizes_ref, s_ref, e_ref, o_ref):
    st = starts_ref[...]
    sz = sizes_ref[...]
    end = st + sz
    p = jnp.minimum(((st + BLOCK - 1) // BLOCK) * BLOCK, end)
    q = jnp.maximum((end // BLOCK) * BLOCK, p)
    blk = jax.lax.broadcasted_iota(jnp.int32, (N_MOLS, NBLKS), 1) * BLOCK
    w = jnp.where((blk >= p) & (blk + BLOCK <= q), 1.0, 0.0)
    interior = jnp.dot(w, s_ref[...], preferred_element_type=jnp.float32)
    o_ref[...] = (interior + e_ref[...]) / sz.astype(jnp.float32)


def _tc2_join(starts2d, sizes2d, block_sums, edges):
    return pl.pallas_call(
        _tc2_body,
        out_shape=jax.ShapeDtypeStruct((N_MOLS, D), jnp.float32),
    )(starts2d, sizes2d, block_sums, edges)


def kernel(atom_hiddens, a_scope):
    starts = a_scope[:, 0]
    sizes = a_scope[:, 1]
    edges = _sc_edges(atom_hiddens, starts, sizes).reshape(N_MOLS, D)
    block_sums = _tc1_block_sums(atom_hiddens)
    return _tc2_join(starts.reshape(N_MOLS, 1), sizes.reshape(N_MOLS, 1),
                     block_sums, edges)


# speedup vs baseline: 1.0770x; 1.0770x over previous
import functools

import jax
import jax.numpy as jnp
from jax import lax
from jax.experimental import pallas as pl
from jax.experimental.pallas import tpu as pltpu
from jax.experimental.pallas import tpu_sc as plsc

N_TOKENS = 32768
D = 128
N_MOLS = 16
L = 16
NV = D // L
CH = 128
NBUF = 4


def _sc_body(ah, starts_hbm, sizes_hbm, out, buf, acc_v, t0, t1, scope_v,
             shared, *sems):
    c = lax.axis_index("c")
    s = lax.axis_index("s")

    pltpu.sync_copy(starts_hbm, scope_v.at[0, pl.ds(0, N_MOLS)])
    pltpu.sync_copy(sizes_hbm, scope_v.at[1, pl.ds(0, N_MOLS)])

    def _at(row, idx):
        return scope_v[row, pl.ds(idx, L)][0]

    m = c * 8 + s // 2
    h = s % 2
    start = _at(0, m)
    size = _at(1, m)
    half = size // 2
    a = start + h * half
    b = a + half + h * (size - 2 * half)
    a8 = (a // 8) * 8
    nch = (b - a8 + CH - 1) // CH

    def _offc(k):
        return jnp.minimum(a8 + k * CH, N_TOKENS - CH)

    def _dma_start(k, buf_ref, sem):
        @pl.when(k < nch)
        def _():
            pltpu.async_copy(ah.at[pl.ds(_offc(k), CH), :], buf_ref, sem)

    def _dma_wait(k, buf_ref, sem):
        @pl.when(k < nch)
        def _():
            pltpu.make_async_copy(ah.at[pl.ds(0, CH), :], buf_ref, sem).wait()

    def _accumulate(k, buf_ref, carry):
        off = a8 + k * CH
        rel_lo = jnp.maximum(a, off) - _offc(k)
        rel_hi = jnp.minimum(b, off + CH) - _offc(k)

        def row_body(r, acc):
            return tuple(acc[kk] + buf_ref[r, pl.ds(kk * L, L)]
                         for kk in range(NV))

        return plsc.parallel_loop(rel_lo, rel_hi, step=1, unroll=8,
                                  carry=carry)(row_body)

    for bslot in range(NBUF):
        _dma_start(bslot, buf.at[bslot], sems[bslot])

    def ring_body(j, carry):
        k0 = NBUF * j
        for bslot in range(NBUF):
            k = k0 + bslot
            _dma_wait(k, buf.at[bslot], sems[bslot])
            carry = _accumulate(k, buf.at[bslot], carry)
            _dma_start(k + NBUF, buf.at[bslot], sems[bslot])
        return carry

    zeros = tuple(jnp.zeros((L,), jnp.float32) for _ in range(NV))
    accs = lax.fori_loop(0, (nch + NBUF - 1) // NBUF, ring_body, zeros)
    for k in range(NV):
        acc_v[pl.ds(k * L, L)] = accs[k]

    pltpu.sync_copy(acc_v, shared.at[s])
    plsc.subcore_barrier()

    @pl.when(s < 8)
    def _():
        pltpu.sync_copy(shared.at[2 * s], t0)
        pltpu.sync_copy(shared.at[2 * s + 1], t1)
        mrow = c * 8 + s
        szvec = jnp.full((L,), _at(1, mrow)).astype(jnp.float32)
        for k in range(NV):
            sl = pl.ds(k * L, L)
            t0[sl] = (t0[sl] + t1[sl]) / szvec
        pltpu.sync_copy(t0, out.at[pl.ds(mrow * D, D)])


_seg_mean = functools.partial(
    pl.kernel,
    out_type=jax.ShapeDtypeStruct((N_MOLS * D,), jnp.float32),
    mesh=plsc.VectorSubcoreMesh(core_axis_name="c", subcore_axis_name="s"),
    compiler_params=pltpu.CompilerParams(use_tc_tiling_on_sc=False),
    scratch_types=[
        pltpu.VMEM((NBUF, CH, D), jnp.float32),
        pltpu.VMEM((D,), jnp.float32),
        pltpu.VMEM((D,), jnp.float32),
        pltpu.VMEM((D,), jnp.float32),
        pltpu.VMEM((2, 2 * L), jnp.int32),
        pltpu.VMEM_SHARED((N_MOLS, D), jnp.float32),
    ] + [pltpu.SemaphoreType.DMA] * NBUF,
)(_sc_body)


def kernel(atom_hiddens, a_scope):
    starts = a_scope[:, 0]
    sizes = a_scope[:, 1]
    return _seg_mean(atom_hiddens, starts, sizes).reshape(N_MOLS, D)


# speedup vs baseline: 1.0830x; 1.0055x over previous
import functools

import jax
import jax.numpy as jnp
from jax import lax
from jax.experimental import pallas as pl
from jax.experimental.pallas import tpu as pltpu
from jax.experimental.pallas import tpu_sc as plsc

N_TOKENS = 32768
D = 128
N_MOLS = 16
L = 16
NV = D // L
CH = 128
NBUF = 4


def _sc_body(ah, starts_hbm, sizes_hbm, out, buf, acc_v, t0, t1, scope_v,
             shared, *sems):
    c = lax.axis_index("c")
    s = lax.axis_index("s")

    pltpu.sync_copy(starts_hbm, scope_v.at[0, pl.ds(0, N_MOLS)])
    pltpu.sync_copy(sizes_hbm, scope_v.at[1, pl.ds(0, N_MOLS)])

    def _at(row, idx):
        return scope_v[row, pl.ds(idx, L)][0]

    m = c * 8 + s // 2
    h = s % 2
    start = _at(0, m)
    size = _at(1, m)
    half = size // 2
    a = start + h * half
    b = a + half + h * (size - 2 * half)
    a8 = (a // 8) * 8
    nch = (b - a8 + CH - 1) // CH

    def _offc(k):
        return jnp.minimum(a8 + k * CH, N_TOKENS - CH)

    def _dma_start(k, buf_ref, sem):
        @pl.when(k < nch)
        def _():
            pltpu.async_copy(ah.at[pl.ds(_offc(k), CH), :], buf_ref, sem)

    def _dma_wait(k, buf_ref, sem):
        @pl.when(k < nch)
        def _():
            pltpu.make_async_copy(ah.at[pl.ds(0, CH), :], buf_ref, sem).wait()

    def _accumulate(k, buf_ref, carry):
        off = a8 + k * CH
        rel_lo = jnp.maximum(a, off) - _offc(k)
        rel_hi = jnp.minimum(b, off + CH) - _offc(k)

        def row_body(r, acc):
            return tuple(acc[kk] + buf_ref[r, pl.ds(kk * L, L)]
                         for kk in range(NV))

        return plsc.parallel_loop(rel_lo, rel_hi, step=1, unroll=4,
                                  carry=carry)(row_body)

    for bslot in range(NBUF):
        _dma_start(bslot, buf.at[bslot], sems[bslot])

    def ring_body(j, carry):
        k0 = NBUF * j
        for bslot in range(NBUF):
            k = k0 + bslot
            _dma_wait(k, buf.at[bslot], sems[bslot])
            carry = _accumulate(k, buf.at[bslot], carry)
            _dma_start(k + NBUF, buf.at[bslot], sems[bslot])
        return carry

    zeros = tuple(jnp.zeros((L,), jnp.float32) for _ in range(NV))
    accs = lax.fori_loop(0, (nch + NBUF - 1) // NBUF, ring_body, zeros)
    for k in range(NV):
        acc_v[pl.ds(k * L, L)] = accs[k]

    pltpu.sync_copy(acc_v, shared.at[s])
    plsc.subcore_barrier()

    @pl.when(s < 8)
    def _():
        pltpu.sync_copy(shared.at[2 * s], t0)
        pltpu.sync_copy(shared.at[2 * s + 1], t1)
        mrow = c * 8 + s
        szvec = jnp.full((L,), _at(1, mrow)).astype(jnp.float32)
        for k in range(NV):
            sl = pl.ds(k * L, L)
            t0[sl] = (t0[sl] + t1[sl]) / szvec
        pltpu.sync_copy(t0, out.at[pl.ds(mrow * D, D)])


_seg_mean = functools.partial(
    pl.kernel,
    out_type=jax.ShapeDtypeStruct((N_MOLS * D,), jnp.float32),
    mesh=plsc.VectorSubcoreMesh(core_axis_name="c", subcore_axis_name="s"),
    compiler_params=pltpu.CompilerParams(use_tc_tiling_on_sc=False),
    scratch_types=[
        pltpu.VMEM((NBUF, CH, D), jnp.float32),
        pltpu.VMEM((D,), jnp.float32),
        pltpu.VMEM((D,), jnp.float32),
        pltpu.VMEM((D,), jnp.float32),
        pltpu.VMEM((2, 2 * L), jnp.int32),
        pltpu.VMEM_SHARED((N_MOLS, D), jnp.float32),
    ] + [pltpu.SemaphoreType.DMA] * NBUF,
)(_sc_body)


def kernel(atom_hiddens, a_scope):
    starts = a_scope[:, 0]
    sizes = a_scope[:, 1]
    return _seg_mean(atom_hiddens, starts, sizes).reshape(N_MOLS, D)
